# baseline (device time: 51690 ns/iter reference)
import jax
import jax.numpy as jnp
from jax import lax
from jax.experimental import pallas as pl
from jax.experimental.pallas import tpu as pltpu

N_DEV = 4


def kernel(x, dest):
    m_per, n = x.shape
    dest2d = dest.reshape(1, m_per)

    def body(x_ref, dest_ref, out_ref, x_all, dest_all,
             xs_sems, xr_sems, ds_sems, dr_sems):
        my_x = lax.axis_index("x")
        my_y = lax.axis_index("y")
        my_z = lax.axis_index("z")
        left = lax.rem(my_z + (N_DEV - 1), N_DEV)
        right = lax.rem(my_z + 1, N_DEV)

        barrier_sem = pltpu.get_barrier_semaphore()
        for nbr in (left, right):
            pl.semaphore_signal(
                barrier_sem, inc=1,
                device_id=(my_x, my_y, nbr),
                device_id_type=pl.DeviceIdType.MESH,
            )
        pl.semaphore_wait(barrier_sem, 2)

        x_all[pl.ds(my_z, 1)] = x_ref[:][None].astype(jnp.bfloat16)
        dest_all[pl.ds(my_z, 1)] = dest_ref[:]

        for h in range(N_DEV - 1):
            slot = lax.rem(my_z + (N_DEV - h), N_DEV)
            x_rdma = pltpu.make_async_remote_copy(
                src_ref=x_all.at[pl.ds(slot, 1)],
                dst_ref=x_all.at[pl.ds(slot, 1)],
                send_sem=xs_sems.at[h],
                recv_sem=xr_sems.at[h],
                device_id=(my_x, my_y, right),
                device_id_type=pl.DeviceIdType.MESH,
            )
            d_rdma = pltpu.make_async_remote_copy(
                src_ref=dest_all.at[pl.ds(slot, 1)],
                dst_ref=dest_all.at[pl.ds(slot, 1)],
                send_sem=ds_sems.at[h],
                recv_sem=dr_sems.at[h],
                device_id=(my_x, my_y, right),
                device_id_type=pl.DeviceIdType.MESH,
            )
            x_rdma.start()
            d_rdma.start()
            x_rdma.wait()
            d_rdma.wait()

        d_all = dest_all[:, :]
        mask = (d_all == my_z)
        m0 = mask.astype(jnp.int32)

        m_incl = m0
        k = 1
        while k < m_per:
            shifted = jnp.concatenate(
                [jnp.zeros((N_DEV, k), jnp.int32), m_incl[:, : m_per - k]],
                axis=1,
            )
            m_incl = m_incl + shifted
            k *= 2

        t = m_incl[:, m_per - 1 : m_per]
        t_excl = (
            jnp.concatenate([jnp.zeros((1, 1), jnp.int32), t[:3]], axis=0)
            + jnp.concatenate([jnp.zeros((2, 1), jnp.int32), t[:2]], axis=0)
            + jnp.concatenate([jnp.zeros((3, 1), jnp.int32), t[:1]], axis=0)
        )
        pos = m_incl - m0 + t_excl

        iota_j = lax.broadcasted_iota(jnp.int32, (m_per, m_per), 0)
        acc = jnp.zeros((m_per, n), jnp.float32)
        for s in range(N_DEV):
            p = jnp.logical_and(
                pos[s : s + 1, :] == iota_j, mask[s : s + 1, :]
            ).astype(jnp.bfloat16)
            acc = acc + lax.dot_general(
                p, x_all[s],
                (((1,), (0,)), ((), ())),
                preferred_element_type=jnp.float32,
            )
        out_ref[:] = acc

    return pl.pallas_call(
        body,
        out_shape=jax.ShapeDtypeStruct((m_per, n), jnp.float32),
        in_specs=[
            pl.BlockSpec(memory_space=pltpu.VMEM),
            pl.BlockSpec(memory_space=pltpu.VMEM),
        ],
        out_specs=pl.BlockSpec(memory_space=pltpu.VMEM),
        scratch_shapes=[
            pltpu.VMEM((N_DEV, m_per, n), jnp.bfloat16),
            pltpu.VMEM((N_DEV, m_per), jnp.int32),
            pltpu.SemaphoreType.DMA((N_DEV - 1,)),
            pltpu.SemaphoreType.DMA((N_DEV - 1,)),
            pltpu.SemaphoreType.DMA((N_DEV - 1,)),
            pltpu.SemaphoreType.DMA((N_DEV - 1,)),
        ],
        compiler_params=pltpu.CompilerParams(collective_id=0),
    )(x, dest2d)


# device time: 22339 ns/iter; 2.3139x vs baseline; 2.3139x over previous
import jax
import jax.numpy as jnp
from jax import lax
from jax.experimental import pallas as pl
from jax.experimental.pallas import tpu as pltpu

N_DEV = 4
CAP = 288


def kernel(x, dest):
    m_per, n = x.shape
    dest2d = dest.reshape(1, m_per)

    def body(x_ref, dest_ref, out_ref, dest_all, pack, recv_buf,
             d_send_sems, d_recv_sems, x_send_sems, x_recv_sems):
        my_x = lax.axis_index("x")
        my_y = lax.axis_index("y")
        my_z = lax.axis_index("z")

        barrier_sem = pltpu.get_barrier_semaphore()
        for k in range(1, N_DEV):
            p = lax.rem(my_z + k, N_DEV)
            pl.semaphore_signal(
                barrier_sem, inc=1,
                device_id=(my_x, my_y, p),
                device_id_type=pl.DeviceIdType.MESH,
            )
        pl.semaphore_wait(barrier_sem, N_DEV - 1)

        dest_all[pl.ds(my_z, 1)] = dest_ref[:]
        d_rdmas = []
        for k in range(1, N_DEV):
            p = lax.rem(my_z + k, N_DEV)
            r = pltpu.make_async_remote_copy(
                src_ref=dest_all.at[pl.ds(my_z, 1)],
                dst_ref=dest_all.at[pl.ds(my_z, 1)],
                send_sem=d_send_sems.at[k - 1],
                recv_sem=d_recv_sems.at[3 - k],
                device_id=(my_x, my_y, p),
                device_id_type=pl.DeviceIdType.MESH,
            )
            r.start()
            d_rdmas.append(r)

        x_bf = x_ref[:].astype(jnp.bfloat16)
        dloc = dest_ref[:]
        iota_d = lax.broadcasted_iota(jnp.int32, (N_DEV, m_per), 0)
        masks = (iota_d == dloc).astype(jnp.int32)

        def prefix_lanes(m):
            k = 1
            while k < m_per:
                m = m + jnp.concatenate(
                    [jnp.zeros((N_DEV, k), jnp.int32), m[:, : m_per - k]],
                    axis=1,
                )
                k *= 2
            return m

        pos_l = prefix_lanes(masks) - masks
        iota_r = lax.broadcasted_iota(jnp.int32, (CAP, m_per), 0)
        for d in range(N_DEV):
            s_mat = jnp.logical_and(
                pos_l[d : d + 1, :] == iota_r, masks[d : d + 1, :] != 0
            ).astype(jnp.bfloat16)
            pack[d] = lax.dot_general(
                s_mat, x_bf, (((1,), (0,)), ((), ())),
                preferred_element_type=jnp.float32,
            ).astype(jnp.bfloat16)

        x_rdmas = []
        for k in range(1, N_DEV):
            p = lax.rem(my_z + k, N_DEV)
            r = pltpu.make_async_remote_copy(
                src_ref=pack.at[pl.ds(p, 1)],
                dst_ref=recv_buf.at[pl.ds(my_z, 1)],
                send_sem=x_send_sems.at[k - 1],
                recv_sem=x_recv_sems.at[3 - k],
                device_id=(my_x, my_y, p),
                device_id_type=pl.DeviceIdType.MESH,
            )
            r.start()
            x_rdmas.append(r)

        for j in range(1, N_DEV):
            s = lax.rem(my_z + j, N_DEV)
            pltpu.make_async_remote_copy(
                src_ref=dest_all.at[pl.ds(my_z, 1)],
                dst_ref=dest_all.at[pl.ds(s, 1)],
                send_sem=d_send_sems.at[0],
                recv_sem=d_recv_sems.at[j - 1],
                device_id=(my_x, my_y, my_z),
                device_id_type=pl.DeviceIdType.MESH,
            ).wait_recv()

        mg = (dest_all[:, :] == my_z).astype(jnp.int32)
        incl_g = prefix_lanes(mg)
        t = incl_g[:, m_per - 1 : m_per]
        t_excl = (
            jnp.concatenate([jnp.zeros((1, 1), jnp.int32), t[:3]], axis=0)
            + jnp.concatenate([jnp.zeros((2, 1), jnp.int32), t[:2]], axis=0)
            + jnp.concatenate([jnp.zeros((3, 1), jnp.int32), t[:1]], axis=0)
        )

        iota4 = lax.broadcasted_iota(jnp.int32, (N_DEV, 1), 0)
        rowj = lax.broadcasted_iota(jnp.int32, (m_per, CAP), 0)
        colr = lax.broadcasted_iota(jnp.int32, (m_per, CAP), 1)

        def contrib(s, blk):
            off = jnp.sum(jnp.where(iota4 == s, t_excl, 0))
            cnt = jnp.sum(jnp.where(iota4 == s, t, 0))
            q = jnp.logical_and(
                (rowj - colr) == off, colr < cnt
            ).astype(jnp.bfloat16)
            return lax.dot_general(
                q, blk, (((1,), (0,)), ((), ())),
                preferred_element_type=jnp.float32,
            )

        acc = contrib(my_z, pack[pl.ds(my_z, 1)][0])
        for j in range(1, N_DEV):
            s = lax.rem(my_z + j, N_DEV)
            pltpu.make_async_remote_copy(
                src_ref=pack.at[pl.ds(my_z, 1)],
                dst_ref=recv_buf.at[pl.ds(s, 1)],
                send_sem=x_send_sems.at[0],
                recv_sem=x_recv_sems.at[j - 1],
                device_id=(my_x, my_y, my_z),
                device_id_type=pl.DeviceIdType.MESH,
            ).wait_recv()
            acc = acc + contrib(s, recv_buf[pl.ds(s, 1)][0])
        out_ref[:] = acc

        for r in d_rdmas:
            r.wait_send()
        for r in x_rdmas:
            r.wait_send()

    return pl.pallas_call(
        body,
        out_shape=jax.ShapeDtypeStruct((m_per, n), jnp.float32),
        in_specs=[
            pl.BlockSpec(memory_space=pltpu.VMEM),
            pl.BlockSpec(memory_space=pltpu.VMEM),
        ],
        out_specs=pl.BlockSpec(memory_space=pltpu.VMEM),
        scratch_shapes=[
            pltpu.VMEM((N_DEV, m_per), jnp.int32),
            pltpu.VMEM((N_DEV, CAP, n), jnp.bfloat16),
            pltpu.VMEM((N_DEV, CAP, n), jnp.bfloat16),
            pltpu.SemaphoreType.DMA((N_DEV - 1,)),
            pltpu.SemaphoreType.DMA((N_DEV - 1,)),
            pltpu.SemaphoreType.DMA((N_DEV - 1,)),
            pltpu.SemaphoreType.DMA((N_DEV - 1,)),
        ],
        compiler_params=pltpu.CompilerParams(collective_id=0),
    )(x, dest2d)
